# baseline (speedup 1.0000x reference)
import jax
import jax.numpy as jnp
from jax.experimental import pallas as pl

DIM = 64
N_EMBED = 1024
B, H, W = 16, 32, 32
NTOK = H * W
TOTAL = B * DIM * H * W


BPS = 16
GRID = B // BPS


def _vq_body(x_ref, e_ref, q_ref, ind_ref, dsum_ref):
    g = pl.program_id(0)
    e = e_ref[...]

    et = e.T
    e2t = jnp.sum(et * et, axis=1, keepdims=True)

    part = jnp.zeros((1, 1), jnp.float32)
    for i in range(BPS):
        x = x_ref[i]
        x2 = jnp.sum(x * x, axis=0, keepdims=True)
        prodt = jnp.dot(et, x, preferred_element_type=jnp.float32)
        dist = x2 - 2.0 * prodt + e2t
        ind = jnp.argmin(dist, axis=0).astype(jnp.int32)

        code_ids = jax.lax.broadcasted_iota(jnp.int32, (N_EMBED, NTOK), 0)
        onehot = (code_ids == ind[None, :]).astype(jnp.float32)
        q = jax.lax.dot(e, onehot,
                        preferred_element_type=jnp.float32)

        q_ref[i] = q
        ind_ref[i, 0] = ind
        part = part + jnp.sum((q - x) ** 2).reshape(1, 1)

    @pl.when(g == 0)
    def _init():
        dsum_ref[...] = jnp.zeros((1, 1), jnp.float32)

    dsum_ref[...] += part

    @pl.when(g == GRID - 1)
    def _finish():
        dsum_ref[...] = dsum_ref[...] / TOTAL


def kernel(input, embeddings):
    x = input.reshape(B, DIM, NTOK)
    q, ind, dsum = pl.pallas_call(
        _vq_body,
        grid=(GRID,),
        in_specs=[
            pl.BlockSpec((BPS, DIM, NTOK), lambda g: (g, 0, 0)),
            pl.BlockSpec((DIM, N_EMBED), lambda g: (0, 0)),
        ],
        out_specs=[
            pl.BlockSpec((BPS, DIM, NTOK), lambda g: (g, 0, 0)),
            pl.BlockSpec((BPS, 1, NTOK), lambda g: (g, 0, 0)),
            pl.BlockSpec((1, 1), lambda g: (0, 0)),
        ],
        out_shape=[
            jax.ShapeDtypeStruct((B, DIM, NTOK), jnp.float32),
            jax.ShapeDtypeStruct((B, 1, NTOK), jnp.int32),
            jax.ShapeDtypeStruct((1, 1), jnp.float32),
        ],
    )(x, embeddings)
    quantize_st = q.reshape(B, DIM, H, W)
    embed_ind = ind.reshape(B, H, W)
    diff = dsum.reshape(())
    return quantize_st, diff, embed_ind


# speedup vs baseline: 1.0887x; 1.0887x over previous
import jax
import jax.numpy as jnp
from jax.experimental import pallas as pl

DIM = 64
N_EMBED = 1024
B, H, W = 16, 32, 32
NTOK = H * W
TOTAL = B * DIM * H * W
AUG = DIM + 8

BPS = 8
GRID = B // BPS


def _vq_body(x_ref, e_ref, ea_ref, q_ref, ind_ref, dsum_ref):
    g = pl.program_id(0)
    e = e_ref[...]
    e_aug = ea_ref[...]

    et = e.T
    e2t = jnp.sum(et * et, axis=1, keepdims=True)

    part = jnp.zeros((1, 1), jnp.float32)
    for i in range(BPS):
        x = x_ref[i]
        x2 = jnp.sum(x * x, axis=0, keepdims=True)
        prodt = jnp.dot(et, x, preferred_element_type=jnp.float32)
        dist = x2 - 2.0 * prodt + e2t
        m = jnp.min(dist, axis=0, keepdims=True)

        onehot = (dist == m).astype(jnp.bfloat16)
        qa = jax.lax.dot(e_aug, onehot,
                         preferred_element_type=jnp.float32)
        count = qa[DIM:DIM + 1]
        rec = 1.0 / count
        ind_f = (qa[DIM + 1:DIM + 2] * 32.0 + qa[DIM + 2:DIM + 3]) * rec
        ind = (ind_f + 0.5).astype(jnp.int32)[0]
        q = qa[0:DIM] * rec

        q_ref[i] = q
        ind_ref[i, 0] = ind
        part = part + jnp.sum((q - x) ** 2).reshape(1, 1)

    @pl.when(g == 0)
    def _init():
        dsum_ref[...] = jnp.zeros((1, 1), jnp.float32)

    dsum_ref[...] += part

    @pl.when(g == GRID - 1)
    def _finish():
        dsum_ref[...] = dsum_ref[...] / TOTAL


def kernel(input, embeddings):
    x = input.reshape(B, DIM, NTOK)
    j = jnp.arange(N_EMBED, dtype=jnp.float32)
    extra = jnp.zeros((8, N_EMBED), jnp.float32)
    extra = extra.at[0].set(1.0).at[1].set(jnp.floor(j / 32.0)).at[2].set(j % 32.0)
    e_aug = jnp.concatenate([embeddings, extra], axis=0).astype(jnp.bfloat16)
    q, ind, dsum = pl.pallas_call(
        _vq_body,
        grid=(GRID,),
        in_specs=[
            pl.BlockSpec((BPS, DIM, NTOK), lambda g: (g, 0, 0)),
            pl.BlockSpec((DIM, N_EMBED), lambda g: (0, 0)),
            pl.BlockSpec((AUG, N_EMBED), lambda g: (0, 0)),
        ],
        out_specs=[
            pl.BlockSpec((BPS, DIM, NTOK), lambda g: (g, 0, 0)),
            pl.BlockSpec((BPS, 1, NTOK), lambda g: (g, 0, 0)),
            pl.BlockSpec((1, 1), lambda g: (0, 0)),
        ],
        out_shape=[
            jax.ShapeDtypeStruct((B, DIM, NTOK), jnp.float32),
            jax.ShapeDtypeStruct((B, 1, NTOK), jnp.int32),
            jax.ShapeDtypeStruct((1, 1), jnp.float32),
        ],
    )(x, embeddings, e_aug)
    quantize_st = q.reshape(B, DIM, H, W)
    embed_ind = ind.reshape(B, H, W)
    diff = dsum.reshape(())
    return quantize_st, diff, embed_ind


# speedup vs baseline: 1.1126x; 1.0220x over previous
import jax
import jax.numpy as jnp
from jax.experimental import pallas as pl

DIM = 64
N_EMBED = 1024
B, H, W = 16, 32, 32
NTOK = H * W
TOTAL = B * DIM * H * W
AUG = DIM + 8

BPS = 8
GRID = B // BPS


def _vq_body(x_ref, e_ref, ea_ref, q_ref, ind_ref, dsum_ref):
    g = pl.program_id(0)
    e = e_ref[...]
    e_aug = ea_ref[...]

    et = e.T
    e2t = jnp.sum(et * et, axis=1, keepdims=True)
    etm2 = et * -2.0

    part = jnp.zeros((1, 1), jnp.float32)
    for i in range(BPS):
        x = x_ref[i]
        x2 = jnp.sum(x * x, axis=0, keepdims=True)
        prodt = jnp.dot(etm2, x, preferred_element_type=jnp.float32)
        dist = (x2 + prodt) + e2t
        m = jnp.min(dist, axis=0, keepdims=True)

        onehot = (dist == m).astype(jnp.bfloat16)
        qa = jax.lax.dot(e_aug, onehot,
                         preferred_element_type=jnp.float32)
        count = qa[DIM:DIM + 1]
        rec = 1.0 / count
        ind_f = (qa[DIM + 1:DIM + 2] * 32.0 + qa[DIM + 2:DIM + 3]) * rec
        ind = (ind_f + 0.5).astype(jnp.int32)[0]
        q = qa[0:DIM] * rec

        q_ref[i] = q
        ind_ref[i, 0] = ind
        part = part + jnp.sum((q - x) ** 2).reshape(1, 1)

    @pl.when(g == 0)
    def _init():
        dsum_ref[...] = jnp.zeros((1, 1), jnp.float32)

    dsum_ref[...] += part

    @pl.when(g == GRID - 1)
    def _finish():
        dsum_ref[...] = dsum_ref[...] / TOTAL


def kernel(input, embeddings):
    x = input.reshape(B, DIM, NTOK)
    j = jnp.arange(N_EMBED, dtype=jnp.float32)
    extra = jnp.zeros((8, N_EMBED), jnp.float32)
    extra = extra.at[0].set(1.0).at[1].set(jnp.floor(j / 32.0)).at[2].set(j % 32.0)
    e_aug = jnp.concatenate([embeddings, extra], axis=0).astype(jnp.bfloat16)
    q, ind, dsum = pl.pallas_call(
        _vq_body,
        grid=(GRID,),
        in_specs=[
            pl.BlockSpec((BPS, DIM, NTOK), lambda g: (g, 0, 0)),
            pl.BlockSpec((DIM, N_EMBED), lambda g: (0, 0)),
            pl.BlockSpec((AUG, N_EMBED), lambda g: (0, 0)),
        ],
        out_specs=[
            pl.BlockSpec((BPS, DIM, NTOK), lambda g: (g, 0, 0)),
            pl.BlockSpec((BPS, 1, NTOK), lambda g: (g, 0, 0)),
            pl.BlockSpec((1, 1), lambda g: (0, 0)),
        ],
        out_shape=[
            jax.ShapeDtypeStruct((B, DIM, NTOK), jnp.float32),
            jax.ShapeDtypeStruct((B, 1, NTOK), jnp.int32),
            jax.ShapeDtypeStruct((1, 1), jnp.float32),
        ],
    )(x, embeddings, e_aug)
    quantize_st = q.reshape(B, DIM, H, W)
    embed_ind = ind.reshape(B, H, W)
    diff = dsum.reshape(())
    return quantize_st, diff, embed_ind
